# baseline (device time: 2731043 ns/iter reference)
import jax
import jax.numpy as jnp
from jax import lax
from jax.experimental import pallas as pl
from jax.experimental.pallas import tpu as pltpu

N_Z = 2
CAP = 320


def _exchange(x, router):
    t_loc, d = x.shape
    e_loc = router.shape[1]
    t = N_Z * t_loc

    def body(x_ref, r_ref, xfull_ref, rhalves_ref,
             sem_xs, sem_xr, sem_rs, sem_rr):
        my_x = lax.axis_index("x")
        my_y = lax.axis_index("y")
        my_z = lax.axis_index("z")
        peer = (my_x, my_y, 1 - my_z)

        barrier_sem = pltpu.get_barrier_semaphore()
        pl.semaphore_signal(barrier_sem, inc=1, device_id=peer,
                            device_id_type=pl.DeviceIdType.MESH)
        pl.semaphore_wait(barrier_sem, 1)

        rdma_x = pltpu.make_async_remote_copy(
            src_ref=x_ref,
            dst_ref=xfull_ref.at[pl.ds(my_z * t_loc, t_loc), :],
            send_sem=sem_xs, recv_sem=sem_xr,
            device_id=peer, device_id_type=pl.DeviceIdType.MESH)
        rdma_x.start()
        rdma_r = pltpu.make_async_remote_copy(
            src_ref=r_ref,
            dst_ref=rhalves_ref.at[my_z],
            send_sem=sem_rs, recv_sem=sem_rr,
            device_id=peer, device_id_type=pl.DeviceIdType.MESH)
        rdma_r.start()

        xfull_ref[pl.ds(my_z * t_loc, t_loc), :] = x_ref[...]

        @pl.when(my_z == 0)
        def _():
            rhalves_ref[0] = r_ref[...]

        @pl.when(my_z == 1)
        def _():
            rhalves_ref[1] = r_ref[...]

        rdma_r.wait()
        rdma_x.wait()

    return pl.pallas_call(
        body,
        out_shape=(
            jax.ShapeDtypeStruct((t, d), jnp.float32),
            jax.ShapeDtypeStruct((N_Z, d, e_loc), jnp.float32),
        ),
        in_specs=[
            pl.BlockSpec(memory_space=pltpu.VMEM),
            pl.BlockSpec(memory_space=pltpu.VMEM),
        ],
        out_specs=(
            pl.BlockSpec(memory_space=pltpu.VMEM),
            pl.BlockSpec(memory_space=pltpu.VMEM),
        ),
        scratch_shapes=[pltpu.SemaphoreType.DMA] * 4,
        compiler_params=pltpu.CompilerParams(
            collective_id=0,
            vmem_limit_bytes=100 * 1024 * 1024,
        ),
    )(x, router)


def _ffn(xg, ws, W1, W2):
    e_loc, cap, d = xg.shape
    f = W1.shape[2]
    n_chunks = f // d

    def body(xg_ref, ws_ref, w1_any, w2_any, out_ref,
             wchunk_ref, w1b_ref, w2b_ref, sem_w):
        for e in range(e_loc):
            for c in range(n_chunks):
                cp = pltpu.make_async_copy(
                    w1_any.at[e, :, pl.ds(c * d, d)], wchunk_ref, sem_w)
                cp.start()
                cp.wait()
                w1b_ref[:, pl.ds(c * d, d)] = \
                    wchunk_ref[...].astype(jnp.bfloat16)
            for c in range(n_chunks):
                cp = pltpu.make_async_copy(
                    w2_any.at[e, pl.ds(c * d, d), :], wchunk_ref, sem_w)
                cp.start()
                cp.wait()
                w2b_ref[pl.ds(c * d, d), :] = \
                    wchunk_ref[...].astype(jnp.bfloat16)

            xt = xg_ref[e].astype(jnp.bfloat16)
            h = jnp.dot(xt, w1b_ref[...], preferred_element_type=jnp.float32)
            h = jnp.maximum(h, 0.0).astype(jnp.bfloat16)
            y = jnp.dot(h, w2b_ref[...], preferred_element_type=jnp.float32)
            out_ref[e] = y * ws_ref[e]

    return pl.pallas_call(
        body,
        out_shape=jax.ShapeDtypeStruct((e_loc, cap, d), jnp.float32),
        in_specs=[
            pl.BlockSpec(memory_space=pltpu.VMEM),
            pl.BlockSpec(memory_space=pltpu.VMEM),
            pl.BlockSpec(memory_space=pl.ANY),
            pl.BlockSpec(memory_space=pl.ANY),
        ],
        out_specs=pl.BlockSpec(memory_space=pltpu.VMEM),
        scratch_shapes=[
            pltpu.VMEM((d, d), jnp.float32),
            pltpu.VMEM((d, f), jnp.bfloat16),
            pltpu.VMEM((f, d), jnp.bfloat16),
            pltpu.SemaphoreType.DMA,
        ],
        compiler_params=pltpu.CompilerParams(
            vmem_limit_bytes=100 * 1024 * 1024,
        ),
    )(xg, ws, W1, W2)


def _combine(acc):
    t, d = acc.shape
    t_loc = t // N_Z

    def body(acc_ref, out_ref, sem_cs, sem_cr):
        my_x = lax.axis_index("x")
        my_y = lax.axis_index("y")
        my_z = lax.axis_index("z")
        peer = (my_x, my_y, 1 - my_z)

        barrier_sem = pltpu.get_barrier_semaphore()
        pl.semaphore_signal(barrier_sem, inc=1, device_id=peer,
                            device_id_type=pl.DeviceIdType.MESH)
        pl.semaphore_wait(barrier_sem, 1)

        rdma_c = pltpu.make_async_remote_copy(
            src_ref=acc_ref.at[pl.ds((1 - my_z) * t_loc, t_loc), :],
            dst_ref=out_ref,
            send_sem=sem_cs, recv_sem=sem_cr,
            device_id=peer, device_id_type=pl.DeviceIdType.MESH)
        rdma_c.start()
        rdma_c.wait()
        out_ref[...] += acc_ref[pl.ds(my_z * t_loc, t_loc), :]

    return pl.pallas_call(
        body,
        out_shape=jax.ShapeDtypeStruct((t_loc, d), jnp.float32),
        in_specs=[pl.BlockSpec(memory_space=pltpu.VMEM)],
        out_specs=pl.BlockSpec(memory_space=pltpu.VMEM),
        scratch_shapes=[pltpu.SemaphoreType.DMA] * 2,
        compiler_params=pltpu.CompilerParams(
            collective_id=1,
            vmem_limit_bytes=100 * 1024 * 1024,
        ),
    )(acc)


def kernel(x, router, W1, W2):
    t_loc, d = x.shape
    e_loc = W1.shape[0]
    t = N_Z * t_loc

    xfull, rhalves = _exchange(x, router)
    rfull = jnp.concatenate([rhalves[0], rhalves[1]], axis=1)

    my_z = lax.axis_index("z")

    g = jnp.dot(xfull, rfull, preferred_element_type=jnp.float32,
                precision=lax.Precision.HIGHEST)
    m1 = jnp.max(g, axis=1, keepdims=True)
    is1 = g >= m1
    gm = jnp.where(is1, -jnp.inf, g)
    m2 = jnp.max(gm, axis=1, keepdims=True)
    is2 = gm >= m2
    e2w = jnp.exp(m2 - m1)
    wfull = (is1.astype(jnp.float32) + is2.astype(jnp.float32) * e2w) \
        / (1.0 + e2w)
    wloc = jnp.where(my_z == 0, wfull[:, :e_loc], wfull[:, e_loc:])

    order = jnp.argsort(-wloc, axis=0)
    topc = order[:CAP, :]
    wsel = jnp.take_along_axis(wloc, topc, axis=0)
    xg = jnp.take(xfull, topc.T.reshape(-1), axis=0)

    yg = _ffn(xg.reshape(e_loc, CAP, d),
              wsel.T.reshape(e_loc, CAP, 1), W1, W2)
    ygflat = yg.reshape(e_loc * CAP, d)

    rank = jnp.argsort(order, axis=0)
    te1 = jnp.argmax(g, axis=1)
    te2 = jnp.argmax(gm, axis=1)

    def contrib(te):
        eloc = te - my_z * e_loc
        elocc = jnp.clip(eloc, 0, e_loc - 1)
        r = jnp.take_along_axis(rank, elocc[:, None], axis=1)[:, 0]
        valid = (eloc >= 0) & (eloc < e_loc) & (r < CAP)
        pos = jnp.where(valid, elocc * CAP + r, 0)
        return jnp.where(valid[:, None], jnp.take(ygflat, pos, axis=0), 0.0)

    acc = contrib(te1) + contrib(te2)
    return _combine(acc)


# device time: 253486 ns/iter; 10.7739x vs baseline; 10.7739x over previous
import jax
import jax.numpy as jnp
from jax import lax
from jax.experimental import pallas as pl
from jax.experimental.pallas import tpu as pltpu

N_Z = 2
CAP = 320


def kernel(x, router, W1, W2):
    t_loc, d = x.shape
    e_loc, _, f = W1.shape
    t = N_Z * t_loc
    n_chunks = f // d

    def body(x_ref, r_ref, w1_any, w2_any, out_ref,
             xfull_ref, rhalves_ref, wloc_ref, rankm_ref, acc_ref,
             w1c_ref, w2c_ref,
             sem_xs, sem_xr, sem_rs, sem_rr, sem_cs, sem_cr,
             sem_w1, sem_w2):
        my_x = lax.axis_index("x")
        my_y = lax.axis_index("y")
        my_z = lax.axis_index("z")
        peer = (my_x, my_y, 1 - my_z)

        barrier_sem = pltpu.get_barrier_semaphore()
        pl.semaphore_signal(barrier_sem, inc=1, device_id=peer,
                            device_id_type=pl.DeviceIdType.MESH)
        pl.semaphore_wait(barrier_sem, 1)

        rdma_x = pltpu.make_async_remote_copy(
            src_ref=x_ref,
            dst_ref=xfull_ref.at[pl.ds(my_z * t_loc, t_loc), :],
            send_sem=sem_xs, recv_sem=sem_xr,
            device_id=peer, device_id_type=pl.DeviceIdType.MESH)
        rdma_x.start()
        rdma_r = pltpu.make_async_remote_copy(
            src_ref=r_ref, dst_ref=rhalves_ref.at[my_z],
            send_sem=sem_rs, recv_sem=sem_rr,
            device_id=peer, device_id_type=pl.DeviceIdType.MESH)
        rdma_r.start()

        def start_pair(k):
            e, c = divmod(k, n_chunks)
            cp1 = pltpu.make_async_copy(
                w1_any.at[e, :, pl.ds(c * d, d)],
                w1c_ref.at[k % 2], sem_w1.at[k % 2])
            cp1.start()
            cp2 = pltpu.make_async_copy(
                w2_any.at[e, pl.ds(c * d, d), :],
                w2c_ref.at[k % 2], sem_w2.at[k % 2])
            cp2.start()
            return cp1, cp2

        pending = {0: start_pair(0)}

        xfull_ref[pl.ds(my_z * t_loc, t_loc), :] = x_ref[...]

        @pl.when(my_z == 0)
        def _():
            rhalves_ref[0] = r_ref[...]

        @pl.when(my_z == 1)
        def _():
            rhalves_ref[1] = r_ref[...]

        rdma_r.wait()
        rdma_x.wait()

        rfull = jnp.concatenate([rhalves_ref[0], rhalves_ref[1]], axis=1)
        xf = xfull_ref[...]
        g = jnp.dot(xf, rfull, preferred_element_type=jnp.float32,
                    precision=lax.Precision.HIGHEST)
        m1 = jnp.max(g, axis=1, keepdims=True)
        is1 = g >= m1
        gm = jnp.where(is1, -jnp.inf, g)
        m2 = jnp.max(gm, axis=1, keepdims=True)
        is2 = gm >= m2
        e2w = jnp.exp(m2 - m1)
        wfull = (is1.astype(jnp.float32) + is2.astype(jnp.float32) * e2w) \
            / (1.0 + e2w)
        wloc = jnp.where(my_z == 0, wfull[:, :e_loc], wfull[:, e_loc:])
        wloc_ref[...] = wloc
        m = (wloc > 0.0).astype(jnp.float32)

        row_i = lax.broadcasted_iota(jnp.int32, (t, t), 0)
        col_i = lax.broadcasted_iota(jnp.int32, (t, t), 1)
        lst = jnp.where(col_i < row_i, 1.0, 0.0)
        rank = jax.lax.dot_general(
            lst, m, (((1,), (0,)), ((), ())),
            preferred_element_type=jnp.float32)
        rankm_ref[...] = jnp.where(m > 0.0, rank, -1.0)

        cap_i = lax.broadcasted_iota(
            jnp.int32, (t, CAP), 1).astype(jnp.float32)

        for e in range(e_loc):
            gt = jnp.where(rankm_ref[:, e:e + 1] == cap_i, 1.0, 0.0)
            xg = jax.lax.dot_general(
                gt, xfull_ref[...], (((0,), (0,)), ((), ())),
                preferred_element_type=jnp.float32)
            y = None
            for c in range(n_chunks):
                k = e * n_chunks + c
                if k + 1 < e_loc * n_chunks:
                    pending[k + 1] = start_pair(k + 1)
                cp1, cp2 = pending.pop(k)
                cp1.wait()
                cp2.wait()
                h = jnp.dot(xg, w1c_ref[k % 2],
                            preferred_element_type=jnp.float32)
                h = jnp.maximum(h, 0.0)
                yc = jnp.dot(h, w2c_ref[k % 2],
                             preferred_element_type=jnp.float32)
                y = yc if y is None else y + yc
            s = jnp.dot(gt, y, preferred_element_type=jnp.float32)
            contrib = wloc_ref[:, e:e + 1] * s
            if e == 0:
                acc_ref[...] = contrib
            else:
                acc_ref[...] += contrib

        rdma_c = pltpu.make_async_remote_copy(
            src_ref=acc_ref.at[pl.ds((1 - my_z) * t_loc, t_loc), :],
            dst_ref=out_ref,
            send_sem=sem_cs, recv_sem=sem_cr,
            device_id=peer, device_id_type=pl.DeviceIdType.MESH)
        rdma_c.start()
        rdma_c.wait()
        out_ref[...] += acc_ref[pl.ds(my_z * t_loc, t_loc), :]

    return pl.pallas_call(
        body,
        out_shape=jax.ShapeDtypeStruct((t_loc, d), jnp.float32),
        in_specs=[
            pl.BlockSpec(memory_space=pltpu.VMEM),
            pl.BlockSpec(memory_space=pltpu.VMEM),
            pl.BlockSpec(memory_space=pl.ANY),
            pl.BlockSpec(memory_space=pl.ANY),
        ],
        out_specs=pl.BlockSpec(memory_space=pltpu.VMEM),
        scratch_shapes=[
            pltpu.VMEM((t, d), jnp.float32),
            pltpu.VMEM((N_Z, d, e_loc), jnp.float32),
            pltpu.VMEM((t, e_loc), jnp.float32),
            pltpu.VMEM((t, e_loc), jnp.float32),
            pltpu.VMEM((t, d), jnp.float32),
            pltpu.VMEM((2, d, d), jnp.float32),
            pltpu.VMEM((2, d, d), jnp.float32),
            pltpu.SemaphoreType.DMA,
            pltpu.SemaphoreType.DMA,
            pltpu.SemaphoreType.DMA,
            pltpu.SemaphoreType.DMA,
            pltpu.SemaphoreType.DMA,
            pltpu.SemaphoreType.DMA,
            pltpu.SemaphoreType.DMA((2,)),
            pltpu.SemaphoreType.DMA((2,)),
        ],
        compiler_params=pltpu.CompilerParams(
            collective_id=0,
            vmem_limit_bytes=100 * 1024 * 1024,
        ),
    )(x, router, W1, W2)
